# Initial kernel scaffold; baseline (speedup 1.0000x reference)
import functools

import jax
import jax.numpy as jnp
from jax import lax
from jax.experimental import pallas as pl
from jax.experimental.pallas import tpu as pltpu
from jax.experimental.pallas import tpu_sc as plsc

N = 10000
E = 160000
D = 256
HALF = 128

NSC = 2
NT = 16
EPT = E // NT
K = 125
R = EPT // K
ROWS_PER_TILE = N // NT


def _sc_segment_sum(hflat, gidx, dsti, zeros):
    mesh = plsc.VectorSubcoreMesh(core_axis_name="c", subcore_axis_name="s")

    @functools.partial(
        pl.kernel,
        mesh=mesh,
        out_type=jax.ShapeDtypeStruct((NSC, N, HALF), jnp.float32),
        scratch_types=[
            pltpu.VMEM((R, K), jnp.int32),
            pltpu.VMEM((R, K), jnp.int32),
            pltpu.VMEM((K, HALF), jnp.float32),
            pltpu.VMEM_SHARED((N, HALF), jnp.float32),
            pltpu.SemaphoreType.DMA,
        ],
    )
    def k(hflat_hbm, gidx_hbm, dsti_hbm, zeros_hbm, out_hbm,
          gidx_t, dst_t, rows, acc, sem):
        c = lax.axis_index("c")
        s = lax.axis_index("s")
        pltpu.sync_copy(gidx_hbm.at[c, s], gidx_t)
        pltpu.sync_copy(dsti_hbm.at[s], dst_t)
        base = s * ROWS_PER_TILE
        pltpu.sync_copy(zeros_hbm.at[pl.ds(base, ROWS_PER_TILE)],
                        acc.at[pl.ds(base, ROWS_PER_TILE)])
        plsc.subcore_barrier()

        def round_body(r, carry):
            pltpu.async_copy(hflat_hbm.at[gidx_t.at[r]], rows, sem).wait()
            pltpu.sync_copy(rows, acc.at[dst_t.at[r]], add=True)
            return carry

        lax.fori_loop(0, R, round_body, 0)
        plsc.subcore_barrier()
        pltpu.sync_copy(acc.at[pl.ds(base, ROWS_PER_TILE)],
                        out_hbm.at[c, pl.ds(base, ROWS_PER_TILE)])

    return k(hflat, gidx, dsti, zeros)


def _tc_layer_body(agg_ref, h_ref, wr_ref, wt_ref, b_ref, o_ref, *, leaky):
    wr = wr_ref[...]
    acc = lax.dot_general(agg_ref[0], wr[:HALF, :],
                          (((1,), (0,)), ((), ())),
                          preferred_element_type=jnp.float32)
    acc = acc + lax.dot_general(agg_ref[1], wr[HALF:, :],
                                (((1,), (0,)), ((), ())),
                                preferred_element_type=jnp.float32)
    acc = acc + lax.dot_general(h_ref[...], wt_ref[...],
                                (((1,), (0,)), ((), ())),
                                preferred_element_type=jnp.float32)
    acc = acc + b_ref[...]
    if leaky:
        acc = jnp.where(acc > 0, acc, 0.1 * acc)
    o_ref[...] = acc


def _tc_layer(agg2, h, w_rel, w_root, b, leaky):
    bn = 1000
    grid = (N // bn,)
    return pl.pallas_call(
        functools.partial(_tc_layer_body, leaky=leaky),
        grid=grid,
        in_specs=[
            pl.BlockSpec((NSC, bn, HALF), lambda i: (0, i, 0)),
            pl.BlockSpec((bn, D), lambda i: (i, 0)),
            pl.BlockSpec((D, D), lambda i: (0, 0)),
            pl.BlockSpec((D, D), lambda i: (0, 0)),
            pl.BlockSpec((1, D), lambda i: (0, 0)),
        ],
        out_specs=pl.BlockSpec((bn, D), lambda i: (i, 0)),
        out_shape=jax.ShapeDtypeStruct((N, D), jnp.float32),
    )(agg2, h, w_rel, w_root, b.reshape(1, D))


def kernel(x, edge_index, edge_attr, batch,
           W_rel_0, b_rel_0, W_root_0,
           W_rel_1, b_rel_1, W_root_1,
           W_rel_2, b_rel_2, W_root_2,
           W_rel_3, b_rel_3, W_root_3):
    src = edge_index[0]
    dst = edge_index[1]
    gidx = jnp.stack([src * 2, src * 2 + 1]).reshape(NSC, NT, R, K)
    dsti = dst.reshape(NT, R, K)
    zeros = jnp.zeros((N, HALF), jnp.float32)

    params = [
        (W_rel_0, W_root_0, b_rel_0),
        (W_rel_1, W_root_1, b_rel_1),
        (W_rel_2, W_root_2, b_rel_2),
        (W_rel_3, W_root_3, b_rel_3),
    ]
    h = x
    for i, (wr, wt, b) in enumerate(params):
        agg2 = _sc_segment_sum(h.reshape(2 * N, HALF), gidx, dsti, zeros)
        h = _tc_layer(agg2, h, wr, wt, b, leaky=(i < 3))
    return h


# speedup vs baseline: 5.0343x; 5.0343x over previous
import functools

import jax
import jax.numpy as jnp
from jax import lax
from jax.experimental import pallas as pl
from jax.experimental.pallas import tpu as pltpu
from jax.experimental.pallas import tpu_sc as plsc

N = 10000
E = 160000
D = 256
HALF = 128

NSC = 2
NT = 16
EPT = E // NT
K = 125
R = EPT // K
STRIPE_STEP = 624
STRIPE_LEN = 640


def _sc_segment_sum(hflat, gidx, dsti, zeros):
    mesh = plsc.VectorSubcoreMesh(core_axis_name="c", subcore_axis_name="s")

    @functools.partial(
        pl.kernel,
        mesh=mesh,
        out_type=jax.ShapeDtypeStruct((NSC, N, HALF), jnp.float32),
        scratch_types=[
            pltpu.VMEM((R, K), jnp.int32),
            pltpu.VMEM((R, K), jnp.int32),
            pltpu.VMEM((K, HALF), jnp.float32),
            pltpu.VMEM_SHARED((N, HALF), jnp.float32),
            pltpu.SemaphoreType.DMA,
        ],
    )
    def k(hflat_hbm, gidx_hbm, dsti_hbm, zeros_hbm, out_hbm,
          gidx_t, dst_t, rows, acc, sem):
        c = lax.axis_index("c")
        s = lax.axis_index("s")
        pltpu.sync_copy(gidx_hbm.at[c, s], gidx_t)
        pltpu.sync_copy(dsti_hbm.at[s], dst_t)
        base = s * STRIPE_STEP
        pltpu.sync_copy(zeros_hbm.at[pl.ds(base, STRIPE_LEN)],
                        acc.at[pl.ds(base, STRIPE_LEN)])
        plsc.subcore_barrier()

        def round_body(r, carry):
            pltpu.async_copy(hflat_hbm.at[gidx_t.at[r]], rows, sem).wait()
            pltpu.sync_copy(rows, acc.at[dst_t.at[r]], add=True)
            return carry

        lax.fori_loop(0, R, round_body, 0)
        plsc.subcore_barrier()
        pltpu.sync_copy(acc.at[pl.ds(base, STRIPE_LEN)],
                        out_hbm.at[c, pl.ds(base, STRIPE_LEN)])

    return k(hflat, gidx, dsti, zeros)


def _tc_layer_body(agg_ref, h_ref, wr_ref, wt_ref, b_ref, o_ref, *, leaky):
    wr = wr_ref[...]
    acc = lax.dot_general(agg_ref[0], wr[:HALF, :],
                          (((1,), (0,)), ((), ())),
                          preferred_element_type=jnp.float32)
    acc = acc + lax.dot_general(agg_ref[1], wr[HALF:, :],
                                (((1,), (0,)), ((), ())),
                                preferred_element_type=jnp.float32)
    acc = acc + lax.dot_general(h_ref[...], wt_ref[...],
                                (((1,), (0,)), ((), ())),
                                preferred_element_type=jnp.float32)
    acc = acc + b_ref[...]
    if leaky:
        acc = jnp.where(acc > 0, acc, 0.1 * acc)
    o_ref[...] = acc


def _tc_layer(agg2, h, w_rel, w_root, b, leaky):
    bn = 1000
    grid = (N // bn,)
    return pl.pallas_call(
        functools.partial(_tc_layer_body, leaky=leaky),
        grid=grid,
        in_specs=[
            pl.BlockSpec((NSC, bn, HALF), lambda i: (0, i, 0)),
            pl.BlockSpec((bn, D), lambda i: (i, 0)),
            pl.BlockSpec((D, D), lambda i: (0, 0)),
            pl.BlockSpec((D, D), lambda i: (0, 0)),
            pl.BlockSpec((1, D), lambda i: (0, 0)),
        ],
        out_specs=pl.BlockSpec((bn, D), lambda i: (i, 0)),
        out_shape=jax.ShapeDtypeStruct((N, D), jnp.float32),
    )(agg2, h, w_rel, w_root, b.reshape(1, D))


def kernel(x, edge_index, edge_attr, batch,
           W_rel_0, b_rel_0, W_root_0,
           W_rel_1, b_rel_1, W_root_1,
           W_rel_2, b_rel_2, W_root_2,
           W_rel_3, b_rel_3, W_root_3):
    src = edge_index[0]
    dst = edge_index[1]
    gidx = jnp.stack([src * 2, src * 2 + 1]).reshape(NSC, NT, R, K)
    dsti = dst.reshape(NT, R, K)
    zeros = jnp.zeros((N, HALF), jnp.float32)

    params = [
        (W_rel_0, W_root_0, b_rel_0),
        (W_rel_1, W_root_1, b_rel_1),
        (W_rel_2, W_root_2, b_rel_2),
        (W_rel_3, W_root_3, b_rel_3),
    ]
    h = x
    for i, (wr, wt, b) in enumerate(params):
        agg2 = _sc_segment_sum(h.reshape(2 * N, HALF), gidx, dsti, zeros)
        h = _tc_layer(agg2, h, wr, wt, b, leaky=(i < 3))
    return h


# speedup vs baseline: 7.3683x; 1.4636x over previous
import functools

import jax
import jax.numpy as jnp
from jax import lax
from jax.experimental import pallas as pl
from jax.experimental.pallas import tpu as pltpu
from jax.experimental.pallas import tpu_sc as plsc

N = 10000
E = 160000
D = 256
HALF = 128

NSC = 2
NT = 16
EPT = E // NT
K = 125
R = EPT // K
NCH = 2
RC = R // NCH
STRIPE_STEP = 624
STRIPE_LEN = 640


def _sc_segment_sum(hflat, gidx, dsti, zeros):
    mesh = plsc.VectorSubcoreMesh(core_axis_name="c", subcore_axis_name="s")

    @functools.partial(
        pl.kernel,
        mesh=mesh,
        out_type=jax.ShapeDtypeStruct((NSC, N, HALF), jnp.float32),
        scratch_types=[
            pltpu.VMEM((RC, K), jnp.int32),
            pltpu.VMEM((RC, K), jnp.int32),
            pltpu.VMEM((K, HALF), jnp.float32),
            pltpu.VMEM((K, HALF), jnp.float32),
            pltpu.VMEM_SHARED((N, HALF), jnp.float32),
            pltpu.SemaphoreType.DMA,
            pltpu.SemaphoreType.DMA,
        ],
    )
    def k(hflat_hbm, gidx_hbm, dsti_hbm, zeros_hbm, out_hbm,
          gidx_t, dst_t, rows0, rows1, acc, sem0, sem1):
        c = lax.axis_index("c")
        s = lax.axis_index("s")
        base = s * STRIPE_STEP
        pltpu.sync_copy(zeros_hbm.at[pl.ds(base, STRIPE_LEN)],
                        acc.at[pl.ds(base, STRIPE_LEN)])
        plsc.subcore_barrier()

        for ch in range(NCH):
            pltpu.sync_copy(gidx_hbm.at[c, s, pl.ds(ch * RC, RC)], gidx_t)
            pltpu.sync_copy(dsti_hbm.at[s, pl.ds(ch * RC, RC)], dst_t)
            pltpu.async_copy(hflat_hbm.at[gidx_t.at[0]], rows0, sem0)
            pltpu.async_copy(hflat_hbm.at[gidx_t.at[1]], rows1, sem1)

            def round_body(i, carry):
                r = 2 * i
                pltpu.make_async_copy(hflat_hbm.at[gidx_t.at[r]], rows0, sem0).wait()
                pltpu.sync_copy(rows0, acc.at[dst_t.at[r]], add=True)
                pltpu.async_copy(hflat_hbm.at[gidx_t.at[r + 2]], rows0, sem0)
                pltpu.make_async_copy(hflat_hbm.at[gidx_t.at[r + 1]], rows1, sem1).wait()
                pltpu.sync_copy(rows1, acc.at[dst_t.at[r + 1]], add=True)
                pltpu.async_copy(hflat_hbm.at[gidx_t.at[r + 3]], rows1, sem1)
                return carry

            lax.fori_loop(0, RC // 2 - 1, round_body, 0)
            pltpu.make_async_copy(hflat_hbm.at[gidx_t.at[RC - 2]], rows0, sem0).wait()
            pltpu.sync_copy(rows0, acc.at[dst_t.at[RC - 2]], add=True)
            pltpu.make_async_copy(hflat_hbm.at[gidx_t.at[RC - 1]], rows1, sem1).wait()
            pltpu.sync_copy(rows1, acc.at[dst_t.at[RC - 1]], add=True)
        plsc.subcore_barrier()
        pltpu.sync_copy(acc.at[pl.ds(base, STRIPE_LEN)],
                        out_hbm.at[c, pl.ds(base, STRIPE_LEN)])

    return k(hflat, gidx, dsti, zeros)


def _tc_layer_body(agg_ref, h_ref, wr_ref, wt_ref, b_ref, o_ref, *, leaky):
    wr = wr_ref[...]
    acc = lax.dot_general(agg_ref[0], wr[:HALF, :],
                          (((1,), (0,)), ((), ())),
                          preferred_element_type=jnp.float32)
    acc = acc + lax.dot_general(agg_ref[1], wr[HALF:, :],
                                (((1,), (0,)), ((), ())),
                                preferred_element_type=jnp.float32)
    acc = acc + lax.dot_general(h_ref[...], wt_ref[...],
                                (((1,), (0,)), ((), ())),
                                preferred_element_type=jnp.float32)
    acc = acc + b_ref[...]
    if leaky:
        acc = jnp.where(acc > 0, acc, 0.1 * acc)
    o_ref[...] = acc


def _tc_layer(agg2, h, w_rel, w_root, b, leaky):
    bn = 1000
    grid = (N // bn,)
    return pl.pallas_call(
        functools.partial(_tc_layer_body, leaky=leaky),
        grid=grid,
        in_specs=[
            pl.BlockSpec((NSC, bn, HALF), lambda i: (0, i, 0)),
            pl.BlockSpec((bn, D), lambda i: (i, 0)),
            pl.BlockSpec((D, D), lambda i: (0, 0)),
            pl.BlockSpec((D, D), lambda i: (0, 0)),
            pl.BlockSpec((1, D), lambda i: (0, 0)),
        ],
        out_specs=pl.BlockSpec((bn, D), lambda i: (i, 0)),
        out_shape=jax.ShapeDtypeStruct((N, D), jnp.float32),
    )(agg2, h, w_rel, w_root, b.reshape(1, D))


def kernel(x, edge_index, edge_attr, batch,
           W_rel_0, b_rel_0, W_root_0,
           W_rel_1, b_rel_1, W_root_1,
           W_rel_2, b_rel_2, W_root_2,
           W_rel_3, b_rel_3, W_root_3):
    src = edge_index[0]
    dst = edge_index[1]
    gidx = jnp.stack([src * 2, src * 2 + 1]).reshape(NSC, NT, R, K)
    dsti = dst.reshape(NT, R, K)
    zeros = jnp.zeros((N, HALF), jnp.float32)

    params = [
        (W_rel_0, W_root_0, b_rel_0),
        (W_rel_1, W_root_1, b_rel_1),
        (W_rel_2, W_root_2, b_rel_2),
        (W_rel_3, W_root_3, b_rel_3),
    ]
    h = x
    for i, (wr, wt, b) in enumerate(params):
        agg2 = _sc_segment_sum(h.reshape(2 * N, HALF), gidx, dsti, zeros)
        h = _tc_layer(agg2, h, wr, wt, b, leaky=(i < 3))
    return h
